# Initial kernel scaffold; baseline (speedup 1.0000x reference)
import jax
import jax.numpy as jnp
from jax.experimental import pallas as pl


def kernel(inputs, W1, b1, W2, b2, W3, b3, emb, Wa, ba, Wv, bv):
    raise NotImplementedError("write your pallas kernel here")


# speedup vs baseline: 1.6625x; 1.6625x over previous
import functools

import jax
import jax.numpy as jnp
from jax import lax
from jax.experimental import pallas as pl
from jax.experimental.pallas import tpu as pltpu
from jax.experimental.pallas import tpu_sc as plsc

B, S, H, K, A = 4096, 512, 128, 512, 512
BB = 256
NW = 32
BPW = B // NW


def _tables_body(emb_ref, wa_ref, ba_ref, wv_ref, bv_ref,
                 prob_ref, val_ref, embsq_ref):
    emb = emb_ref[...]
    logits = lax.dot_general(
        emb, wa_ref[...], (((1,), (1,)), ((), ())),
        precision=lax.Precision.HIGHEST,
        preferred_element_type=jnp.float32) + ba_ref[...]
    prob_ref[...] = jax.nn.softmax(logits, axis=-1)
    val_ref[...] = jnp.sum(emb * wv_ref[...], axis=1,
                           keepdims=True) + bv_ref[0, 0]
    embsq_ref[...] = jnp.sum(emb * emb, axis=1, keepdims=True)


def _mlp_body(in_ref, w1_ref, b1_ref, w2_ref, b2_ref, w3_ref, b3_ref,
              emb_ref, nhe_ref, idx_ref):
    hp = lax.Precision.HIGHEST

    def lin(x, w_ref, b_ref):
        return lax.dot_general(
            x, w_ref[...], (((1,), (1,)), ((), ())),
            precision=hp, preferred_element_type=jnp.float32) + b_ref[...]

    x = jnp.maximum(lin(in_ref[...], w1_ref, b1_ref), 0.0)
    x = jnp.maximum(lin(x, w2_ref, b2_ref), 0.0)
    x = jnp.maximum(lin(x, w3_ref, b3_ref), 0.0)
    s = lax.dot_general(
        x, emb_ref[...], (((1,), (1,)), ((), ())),
        precision=hp, preferred_element_type=jnp.float32) + nhe_ref[...]
    m = jnp.max(s, axis=1, keepdims=True)
    ks = lax.broadcasted_iota(jnp.int32, (BB, K), 1)
    idx_ref[...] = jnp.min(jnp.where(s == m, ks, K), axis=1, keepdims=True)


def _tc_stage(inputs, W1, b1, W2, b2, W3, b3, emb, Wa, ba, Wv, bv):
    prob_t, val_t, embsq = pl.pallas_call(
        _tables_body,
        out_shape=[
            jax.ShapeDtypeStruct((K, A), jnp.float32),
            jax.ShapeDtypeStruct((K, 1), jnp.float32),
            jax.ShapeDtypeStruct((K, 1), jnp.float32),
        ],
    )(emb, Wa, ba.reshape(1, A), Wv, bv.reshape(1, 1))

    neg_half_embsq = (-0.5) * embsq.reshape(1, K)

    grid = B // BB
    full = lambda shape: pl.BlockSpec(shape, lambda i: (0, 0))
    idx = pl.pallas_call(
        _mlp_body,
        grid=(grid,),
        in_specs=[
            pl.BlockSpec((BB, S), lambda i: (i, 0)),
            full((H, S)), full((1, H)),
            full((H, H)), full((1, H)),
            full((H, H)), full((1, H)),
            full((K, H)), full((1, K)),
        ],
        out_specs=pl.BlockSpec((BB, 1), lambda i: (i, 0)),
        out_shape=jax.ShapeDtypeStruct((B, 1), jnp.int32),
    )(inputs, W1, b1.reshape(1, H), W2, b2.reshape(1, H),
      W3, b3.reshape(1, H), emb, neg_half_embsq)
    return prob_t, val_t, idx


def _gather_body(prob_hbm, vtab_hbm, idx_hbm, act_hbm, val_hbm,
                 idx_v, rows_v, val_v, sem, sem2):
    wid = lax.axis_index("s") * 2 + lax.axis_index("c")
    base = wid * BPW
    pltpu.sync_copy(idx_hbm.at[pl.ds(base, BPW)], idx_v)
    cp = pltpu.async_copy(prob_hbm.at[idx_v], rows_v, sem)
    cpv = pltpu.async_copy(vtab_hbm.at[idx_v], val_v, sem2)
    cpv.wait()
    pltpu.sync_copy(val_v, val_hbm.at[pl.ds(base, BPW)])
    cp.wait()
    pltpu.sync_copy(rows_v, act_hbm.at[pl.ds(base, BPW)])


@functools.cache
def _gather_call():
    return functools.partial(
        pl.kernel,
        mesh=plsc.VectorSubcoreMesh(core_axis_name="c", subcore_axis_name="s"),
        out_type=[
            jax.ShapeDtypeStruct((B, A), jnp.float32),
            jax.ShapeDtypeStruct((B,), jnp.float32),
        ],
        scratch_types=[
            pltpu.VMEM((BPW,), jnp.int32),
            pltpu.VMEM((BPW, A), jnp.float32),
            pltpu.VMEM((BPW,), jnp.float32),
            pltpu.SemaphoreType.DMA,
            pltpu.SemaphoreType.DMA,
        ],
    )(_gather_body)


def kernel(inputs, W1, b1, W2, b2, W3, b3, emb, Wa, ba, Wv, bv):
    prob_t, val_t, idx = _tc_stage(
        inputs, W1, b1, W2, b2, W3, b3, emb, Wa, ba, Wv, bv)
    actions_prob, value = _gather_call()(
        prob_t, val_t.reshape(K), idx.reshape(B))
    return actions_prob, value.reshape(B, 1)
